# baseline (speedup 1.0000x reference)
import functools

import jax
import jax.numpy as jnp
from jax import lax
from jax.experimental import pallas as pl
from jax.experimental.pallas import tpu as pltpu
from jax.experimental.pallas import tpu_sc as plsc

N = 10000
E = 320000
D = 128
H = 128

NC = 2
NS = 16
NW = NC * NS
BLK = 128
NBPW = 80
NBLK = NW * NBPW
E2 = NBLK * BLK
EWOFF = 8
STRIPE = 624
TAIL = N - NS * STRIPE

_SQRT1_2 = 0.7071067811865476


def _gelu(t):
    return 0.5 * t * (1.0 + lax.erf(t * _SQRT1_2))


def _mm2_body(h_ref, wa_ref, wb_ref, ya_ref, yb_ref):
    h = h_ref[...]
    ya_ref[...] = jnp.dot(h, wa_ref[...], preferred_element_type=jnp.float32)
    yb_ref[...] = jnp.dot(h, wb_ref[...], preferred_element_type=jnp.float32)


def _mm2(h, wa, wb):
    return pl.pallas_call(
        _mm2_body,
        out_shape=[jax.ShapeDtypeStruct((N, H), jnp.float32),
                   jax.ShapeDtypeStruct((N, H), jnp.float32)],
    )(h, wa, wb)


def _post_body(p_ref, r_ref, br_ref, g_ref, b_ref, out_ref):
    t = p_ref[:N, :] + p_ref[N:, :] + r_ref[...] + br_ref[...]
    g = _gelu(t)
    mu = jnp.mean(g, axis=0, keepdims=True)
    d0 = g - mu
    var = jnp.mean(d0 * d0, axis=0, keepdims=True)
    out_ref[...] = d0 / jnp.sqrt(var + 1e-5) * g_ref[...] + b_ref[...]


def _post(p, r, br, gamma, beta):
    return pl.pallas_call(
        _post_body,
        out_shape=jax.ShapeDtypeStruct((N, H), jnp.float32),
    )(p, r, br.reshape(1, H), gamma.reshape(1, H), beta.reshape(1, H))


_FING = 20
_FINB = NBLK // _FING


def _fin_body(ef_ref, b1_ref, w2_ref, b2_ref, out_ref):
    t = ef_ref[...] + b1_ref[...]
    g = _gelu(t)
    v = jnp.sum(g * w2_ref[...], axis=2)
    out_ref[...] = 1.0 / (1.0 + jnp.exp(-(v + b2_ref[...])))


def _fin(ef, b1, w2, b2):
    return pl.pallas_call(
        _fin_body,
        grid=(_FING,),
        in_specs=[
            pl.BlockSpec((_FINB, BLK, H), lambda i: (i, 0, 0)),
            pl.BlockSpec((1, 1, H), lambda i: (0, 0, 0)),
            pl.BlockSpec((1, 1, H), lambda i: (0, 0, 0)),
            pl.BlockSpec((1, 1), lambda i: (0, 0)),
        ],
        out_specs=pl.BlockSpec((_FINB, BLK), lambda i: (i, 0)),
        out_shape=jax.ShapeDtypeStruct((NBLK, BLK), jnp.float32),
    )(ef, b1.reshape(1, 1, H), w2.reshape(1, 1, H), b2.reshape(1, 1))


_MESH = plsc.VectorSubcoreMesh(core_axis_name="c", subcore_axis_name="s")


PBLK = 16
NPH = NBPW // PBLK


def _sc_agg_body(y_hbm, srcb, dstb, ewb, zeros_hbm, out_hbm,
                 acc, srcv, dstv, ewv, rows, gs0, gs1, ws0, ws1):
    gsems = (gs0, gs1)
    wsems = (ws0, ws1)
    cid = lax.axis_index("c")
    sid = lax.axis_index("s")
    wid = cid * NS + sid
    base = wid * NBPW

    pltpu.sync_copy(zeros_hbm.at[pl.ds(sid * STRIPE, STRIPE)],
                    acc.at[pl.ds(sid * STRIPE, STRIPE)])

    @pl.when(sid == 0)
    def _zero_tail():
        pltpu.sync_copy(zeros_hbm.at[pl.ds(NS * STRIPE, TAIL)],
                        acc.at[pl.ds(NS * STRIPE, TAIL)])

    plsc.subcore_barrier()

    def gstart(i, r):
        pltpu.async_copy(y_hbm.at[srcv.at[i]],
                         rows.at[pl.ds(r * BLK, BLK)], gsems[r])

    def gwait(r):
        pltpu.make_async_copy(y_hbm.at[srcv.at[0]],
                              rows.at[pl.ds(r * BLK, BLK)], gsems[r]).wait()

    def wstart(i, r):
        pltpu.async_copy(rows.at[pl.ds(r * BLK, BLK)],
                         acc.at[dstv.at[i]], wsems[r], add=True)

    def wwait(r):
        pltpu.make_async_copy(rows.at[pl.ds(r * BLK, BLK)],
                              acc.at[dstv.at[0]], wsems[r]).wait()

    def scale(i, r):
        def sgroup(g, carry):
            for j in range(8):
                e = g * 8 + j
                s = plsc.load_gather(
                    ewv, [jnp.full((16,), i + EWOFF, jnp.int32),
                          jnp.full((16,), e, jnp.int32)])
                for c in range(8):
                    sl = pl.ds(c * 16, 16)
                    rows[r * BLK + e, sl] = rows[r * BLK + e, sl] * s
            return carry

        lax.fori_loop(0, 16, sgroup, None)

    def phase(ph, carry):
        @pl.when(ph > 0)
        def _():
            wwait(0)
            wwait(1)

        pbase = base + ph * PBLK
        pltpu.sync_copy(srcb.at[pl.ds(pbase, PBLK)], srcv)
        pltpu.sync_copy(dstb.at[pl.ds(pbase, PBLK)], dstv)
        pltpu.sync_copy(ewb.at[pl.ds(pbase, PBLK)], ewv.at[pl.ds(EWOFF, PBLK)])
        gstart(0, 0)

        def inner(s, c2):
            for r in range(2):
                il = s * 2 + r
                if r == 0:
                    @pl.when(s > 0)
                    def _():
                        wwait(1)

                    gstart(il + 1, 1)
                else:
                    @pl.when(s < (PBLK // 2) - 1)
                    def _():
                        wwait(0)
                        gstart(il + 1, 0)

                gwait(r)
                scale(il, r)
                wstart(il, r)
            return c2

        lax.fori_loop(0, PBLK // 2, inner, None)
        return carry

    lax.fori_loop(0, NPH, phase, None)
    wwait(0)
    wwait(1)
    plsc.subcore_barrier()

    pltpu.sync_copy(acc.at[pl.ds(sid * STRIPE, STRIPE)],
                    out_hbm.at[pl.ds(cid * N + sid * STRIPE, STRIPE)])

    @pl.when(sid == 0)
    def _read_tail():
        pltpu.sync_copy(acc.at[pl.ds(NS * STRIPE, TAIL)],
                        out_hbm.at[pl.ds(cid * N + NS * STRIPE, TAIL)])


@functools.partial(
    pl.kernel,
    out_type=jax.ShapeDtypeStruct((NC * N, H), jnp.float32),
    mesh=_MESH,
    compiler_params=pltpu.CompilerParams(needs_layout_passes=False),
    scratch_types=[
        pltpu.VMEM_SHARED((N, H), jnp.float32),
        pltpu.VMEM((PBLK, BLK), jnp.int32),
        pltpu.VMEM((PBLK, BLK), jnp.int32),
        pltpu.VMEM((PBLK + EWOFF, BLK), jnp.float32),
        pltpu.VMEM((2 * BLK, H), jnp.float32),
        pltpu.SemaphoreType.DMA,
        pltpu.SemaphoreType.DMA,
        pltpu.SemaphoreType.DMA,
        pltpu.SemaphoreType.DMA,
    ],
)
def _sc_agg(y_hbm, srcb, dstb, ewb, zeros_hbm, out_hbm, *rest):
    _sc_agg_body(y_hbm, srcb, dstb, ewb, zeros_hbm, out_hbm, *rest)


def _sc_edge_body(a_hbm, b_hbm, srcb, dstb, out_hbm,
                  srcv, dstv, ra, rb, ga0, ga1, gb0, gb1, ws0, ws1):
    gas = (ga0, ga1)
    gbs = (gb0, gb1)
    wss = (ws0, ws1)
    cid = lax.axis_index("c")
    sid = lax.axis_index("s")
    wid = cid * NS + sid
    base = wid * NBPW

    pltpu.sync_copy(srcb.at[pl.ds(base, NBPW)], srcv)
    pltpu.sync_copy(dstb.at[pl.ds(base, NBPW)], dstv)

    def gstart(i, r):
        pltpu.async_copy(a_hbm.at[srcv.at[i]],
                         ra.at[pl.ds(r * BLK, BLK)], gas[r])
        pltpu.async_copy(b_hbm.at[dstv.at[i]],
                         rb.at[pl.ds(r * BLK, BLK)], gbs[r])

    def gwait(r):
        pltpu.make_async_copy(a_hbm.at[srcv.at[0]],
                              ra.at[pl.ds(r * BLK, BLK)], gas[r]).wait()
        pltpu.make_async_copy(b_hbm.at[dstv.at[0]],
                              rb.at[pl.ds(r * BLK, BLK)], gbs[r]).wait()

    def wstart(i, r):
        pltpu.async_copy(ra.at[pl.ds(r * BLK, BLK)],
                         out_hbm.at[pl.ds((base + i) * BLK, BLK)], wss[r])

    def wwait(r):
        pltpu.make_async_copy(ra.at[pl.ds(r * BLK, BLK)],
                              out_hbm.at[pl.ds(0, BLK)], wss[r]).wait()

    def add(r):
        def agroup(g, carry):
            for j in range(8):
                e = r * BLK + g * 8 + j
                for c in range(8):
                    sl = pl.ds(c * 16, 16)
                    ra[e, sl] = ra[e, sl] + rb[e, sl]
            return carry

        lax.fori_loop(0, 16, agroup, None)

    gstart(0, 0)

    def outer(s, carry):
        for r in range(2):
            i = s * 2 + r
            if r == 0:
                @pl.when(s > 0)
                def _():
                    wwait(1)

                gstart(i + 1, 1)
            else:
                @pl.when(s < (NBPW // 2) - 1)
                def _():
                    wwait(0)
                    gstart(i + 1, 0)

            gwait(r)
            add(r)
            wstart(i, r)
        return carry

    lax.fori_loop(0, NBPW // 2, outer, None)
    wwait(0)
    wwait(1)


@functools.partial(
    pl.kernel,
    out_type=jax.ShapeDtypeStruct((E2, H), jnp.float32),
    mesh=_MESH,
    compiler_params=pltpu.CompilerParams(needs_layout_passes=False),
    scratch_types=[
        pltpu.VMEM((NBPW, BLK), jnp.int32),
        pltpu.VMEM((NBPW, BLK), jnp.int32),
        pltpu.VMEM((2 * BLK, H), jnp.float32),
        pltpu.VMEM((2 * BLK, H), jnp.float32),
        pltpu.SemaphoreType.DMA,
        pltpu.SemaphoreType.DMA,
        pltpu.SemaphoreType.DMA,
        pltpu.SemaphoreType.DMA,
        pltpu.SemaphoreType.DMA,
        pltpu.SemaphoreType.DMA,
    ],
)
def _sc_edge(a_hbm, b_hbm, srcb, dstb, out_hbm, *rest):
    _sc_edge_body(a_hbm, b_hbm, srcb, dstb, out_hbm, *rest)


def kernel(x, edge_index, edge_attr,
           W_rel0, b_rel0, W_root0, gamma0, beta0,
           W_rel1, b_rel1, W_root1, gamma1, beta1,
           W_rel2, b_rel2, W_root2, gamma2, beta2,
           W_mlp1, b_mlp1, W_mlp2, b_mlp2):
    pad = E2 - E
    src = jnp.concatenate([edge_index[0], jnp.zeros((pad,), jnp.int32)])
    dst = jnp.concatenate([edge_index[1], jnp.zeros((pad,), jnp.int32)])
    ew = jnp.concatenate([edge_attr, jnp.zeros((pad,), jnp.float32)])
    srcb = src.reshape(NBLK, BLK)
    dstb = dst.reshape(NBLK, BLK)
    ewb = ew.reshape(NBLK, BLK)
    zeros = jnp.zeros((N, H), jnp.float32)

    h = x
    for (Wr, br, Wt, g, b) in ((W_rel0, b_rel0, W_root0, gamma0, beta0),
                               (W_rel1, b_rel1, W_root1, gamma1, beta1),
                               (W_rel2, b_rel2, W_root2, gamma2, beta2)):
        y, r = _mm2(h, Wr, Wt)
        p = _sc_agg(y, srcb, dstb, ewb, zeros)
        h = _post(p, r, br, g, b)

    a, bm = _mm2(h, W_mlp1[:H], W_mlp1[H:])
    ef = _sc_edge(a, bm, srcb, dstb)
    out = _fin(ef.reshape(NBLK, BLK, H), b_mlp1, W_mlp2, b_mlp2)
    return out.reshape(E2)[:E]
